# baseline (speedup 1.0000x reference)
import jax
import jax.numpy as jnp
from jax.experimental import pallas as pl
from jax.experimental.pallas import tpu as pltpu


_C = 2048


def _body(xf, tab, out, xbuf, tbuf, load_sem, store_sem, tab_sem):
    i = pl.program_id(0)
    T = pl.num_programs(0)

    @pl.when(i == 0)
    def _prologue():
        pltpu.make_async_copy(tab, tbuf, tab_sem).start()
        for t in range(T):
            pltpu.make_async_copy(
                xf.at[pl.ds(t * _C, _C)], xbuf.at[t], load_sem.at[t]).start()
        pltpu.make_async_copy(tab, tbuf, tab_sem).wait()
        tb = tbuf[...]
        norm = jnp.sqrt(jnp.sum(tb * tb, axis=-1, keepdims=True))
        scale = jnp.where(norm > 1.0, 1.0 / (norm + 1e-7), 1.0)
        tbuf[...] = tb * scale

    pltpu.make_async_copy(
        xf.at[pl.ds(i * _C, _C)], xbuf.at[i], load_sem.at[i]).wait()
    xbuf[i] = xbuf[i] + tbuf[...]
    pltpu.make_async_copy(
        xbuf.at[i], out.at[pl.ds(i * _C, _C)], store_sem.at[i]).start()

    @pl.when(i == T - 1)
    def _epilogue():
        for t in range(T):
            pltpu.make_async_copy(
                xbuf.at[t], out.at[pl.ds(t * _C, _C)], store_sem.at[t]).wait()


def kernel(x, table):
    B, S, D = x.shape
    xf = x.reshape(B * S, D)
    T = (B * S) // _C
    out = pl.pallas_call(
        _body,
        grid=(T,),
        in_specs=[
            pl.BlockSpec(memory_space=pl.ANY),
            pl.BlockSpec(memory_space=pl.ANY),
        ],
        out_specs=pl.BlockSpec(memory_space=pl.ANY),
        out_shape=jax.ShapeDtypeStruct((B * S, D), x.dtype),
        scratch_shapes=[
            pltpu.VMEM((T, _C, D), jnp.float32),
            pltpu.VMEM((S, D), jnp.float32),
            pltpu.SemaphoreType.DMA((T,)),
            pltpu.SemaphoreType.DMA((T,)),
            pltpu.SemaphoreType.DMA,
        ],
        compiler_params=pltpu.CompilerParams(
            dimension_semantics=("arbitrary",)),
    )(xf, table)
    return out.reshape(B, S, D)
